# baseline (speedup 1.0000x reference)
import functools

import jax
import jax.numpy as jnp
from jax.experimental import pallas as pl
from jax.experimental.pallas import tpu as pltpu
from jax.experimental.pallas import tpu_sc as plsc

_TN = 256


def _recolor_kernel(tgt_ref, rgb_ref, pred_ref, out_ref,
                    acc_rec, acc_f, acc_cmin, acc_zl, pt_sq):
    i = pl.program_id(0)
    nsteps = pl.num_programs(0)
    f32 = jnp.float32
    bf16 = jnp.bfloat16

    @pl.when(i == 0)
    def _():
        pfull = pred_ref[...]
        pt_sq[0:3, :] = pfull.T
        pt_sq[3:4, :] = jnp.sum(pfull * pfull, axis=1, keepdims=True).T

    t = tgt_ref[...]
    p = pt_sq[0:3, :]
    p_sq = pt_sq[3:4, :]
    rgb1 = jnp.concatenate(
        [rgb_ref[...].T, jnp.ones((1, _TN), f32)], axis=0)

    t_sq = jnp.sum(t * t, axis=1, keepdims=True)
    xp2 = jnp.dot(t.astype(bf16), (p + p).astype(bf16),
                  preferred_element_type=f32)
    d = t_sq + p_sq - xp2

    rowmin = jnp.min(d, axis=1, keepdims=True)
    w = jnp.where(rowmin > 0.0, jax.lax.rsqrt(rowmin), 0.0)
    mask1 = jnp.where(d == rowmin, 1.0, 0.0).astype(bf16)
    rgb1w = (rgb1 * jnp.reshape(w, (1, _TN))).astype(bf16)

    cmax_t = jnp.maximum(jnp.min(d, axis=0, keepdims=True), 0.0)
    m2b = d <= cmax_t
    m2 = jnp.where(m2b, 1.0, 0.0).astype(bf16)

    ridx = jax.lax.broadcasted_iota(jnp.int32, (_TN, 1), 0).astype(f32)
    sel = jnp.where(m2b, ridx, -1.0)
    zl_t = jnp.max(sel, axis=0, keepdims=True)
    base = (i * _TN).astype(f32)
    zl_g = jnp.where((cmax_t <= 0.0) & (zl_t >= 0.0), zl_t + base, -1.0)

    rec_part = jnp.dot(rgb1w, mask1, preferred_element_type=f32)
    f_part = jnp.dot(rgb1.astype(bf16), m2, preferred_element_type=f32)

    @pl.when(i == 0)
    def _():
        acc_rec[...] = rec_part
        acc_f[...] = f_part
        acc_cmin[...] = cmax_t
        acc_zl[...] = zl_g

    @pl.when(i > 0)
    def _():
        acc_rec[...] += rec_part
        acc_zl[...] = jnp.maximum(acc_zl[...], zl_g)
        old = acc_cmin[...]
        better = cmax_t < old
        eq = cmax_t == old
        acc_f[...] = jnp.where(better, f_part,
                               acc_f[...] + jnp.where(eq, f_part, 0.0))
        acc_cmin[...] = jnp.minimum(old, cmax_t)

    @pl.when(i == nsteps - 1)
    def _():
        rec, den = acc_rec[0:3, :], acc_rec[3:4, :]
        fs, fc = acc_f[0:3, :], acc_f[3:4, :]
        zl = acc_zl[...]
        nz = den != 0.0
        r1 = jnp.where(nz, rec / jnp.where(nz, den, 1.0), rec)
        empty = jnp.logical_not(nz) & (zl < 0.0)
        out_ref[0:3, :] = jnp.where(empty, fs / fc, r1)
        out_ref[3:4, :] = zl


def _make_zero_fix(M, N, BW, L, NC):
    mesh = plsc.VectorSubcoreMesh(core_axis_name="c", subcore_axis_name="s")

    @functools.partial(
        pl.kernel, mesh=mesh,
        out_type=jax.ShapeDtypeStruct((3 * M,), jnp.float32),
        compiler_params=pltpu.CompilerParams(needs_layout_passes=False),
        scratch_types=[
            pltpu.VMEM((3 * N,), jnp.float32),
            pltpu.VMEM((4, BW), jnp.float32),
            pltpu.VMEM((3 * BW,), jnp.float32),
        ],
    )
    def zero_fix(pre_hbm, rgb_hbm, out_hbm, rgb_v, pre_v, o_v):
        wid = jax.lax.axis_index("s") * NC + jax.lax.axis_index("c")
        base = wid * BW
        pltpu.sync_copy(rgb_hbm, rgb_v)
        pltpu.sync_copy(pre_hbm.at[:, pl.ds(base, BW)], pre_v)
        lane = jax.lax.iota(jnp.int32, L)
        for j in range(BW // L):
            sl = pl.ds(j * L, L)
            zl = pre_v[3, sl]
            hit = zl >= 0.0
            idx3 = jnp.maximum(zl, 0.0).astype(jnp.int32) * 3
            lidx3 = (lane + (j * L)) * 3
            for c in range(3):
                g = plsc.load_gather(rgb_v, [idx3 + c])
                val = jnp.where(hit, g, pre_v[c, sl])
                plsc.store_scatter(o_v, [lidx3 + c], val)
        pltpu.sync_copy(o_v, out_hbm.at[pl.ds(3 * base, 3 * BW)])

    return zero_fix


def kernel(pred_xyz, tgt_xyz, tgt_rgb, search_range):
    del search_range
    M = pred_xyz.shape[0]
    N = tgt_xyz.shape[0]

    pre = pl.pallas_call(
        _recolor_kernel,
        grid=(N // _TN,),
        in_specs=[
            pl.BlockSpec((_TN, 3), lambda i: (i, 0)),
            pl.BlockSpec((_TN, 3), lambda i: (i, 0)),
            pl.BlockSpec((M, 3), lambda i: (0, 0)),
        ],
        out_specs=pl.BlockSpec((4, M), lambda i: (0, 0)),
        out_shape=jax.ShapeDtypeStruct((4, M), jnp.float32),
        scratch_shapes=[
            pltpu.VMEM((4, M), jnp.float32),
            pltpu.VMEM((4, M), jnp.float32),
            pltpu.VMEM((1, M), jnp.float32),
            pltpu.VMEM((1, M), jnp.float32),
            pltpu.VMEM((4, M), jnp.float32),
        ],
        compiler_params=pltpu.CompilerParams(
            dimension_semantics=("arbitrary",),
        ),
    )(tgt_xyz, tgt_rgb, pred_xyz)

    info = plsc.get_sparse_core_info()
    nw = info.num_cores * info.num_subcores
    zero_fix = _make_zero_fix(M, N, M // nw, info.num_lanes, info.num_cores)
    return zero_fix(pre, tgt_rgb.reshape(3 * N)).reshape(M, 3)


# speedup vs baseline: 1.0119x; 1.0119x over previous
import functools

import jax
import jax.numpy as jnp
from jax.experimental import pallas as pl
from jax.experimental.pallas import tpu as pltpu
from jax.experimental.pallas import tpu_sc as plsc

_TN = 256


def _recolor_kernel(tgt_ref, rgb1_ref, pred_ref, out_ref,
                    acc_rec, acc_f, acc_cmin, acc_zl):
    i = pl.program_id(0)
    nsteps = pl.num_programs(0)
    f32 = jnp.float32
    bf16 = jnp.bfloat16

    t = tgt_ref[...]
    p = pred_ref[...]
    rgb1 = rgb1_ref[...]

    t_sq = jnp.sum(t * t, axis=1, keepdims=True)
    p_sq = jnp.sum(p * p, axis=0, keepdims=True)
    xp2 = jnp.dot(t.astype(bf16), (p + p).astype(bf16),
                  preferred_element_type=f32)
    d = t_sq + p_sq - xp2

    rowmin = jnp.min(d, axis=1, keepdims=True)
    w = jnp.where(rowmin > 0.0, jax.lax.rsqrt(rowmin), 0.0)
    mask1 = jnp.where(d == rowmin, 1.0, 0.0).astype(bf16)
    rgb1w = (rgb1 * jnp.reshape(w, (1, _TN))).astype(bf16)

    cmax_t = jnp.maximum(jnp.min(d, axis=0, keepdims=True), 0.0)
    m2b = d <= cmax_t
    m2 = jnp.where(m2b, 1.0, 0.0).astype(bf16)

    ridx = jax.lax.broadcasted_iota(jnp.int32, (_TN, 1), 0).astype(f32)
    sel = jnp.where(m2b, ridx, -1.0)
    zl_t = jnp.max(sel, axis=0, keepdims=True)
    base = (i * _TN).astype(f32)
    zl_g = jnp.where((cmax_t <= 0.0) & (zl_t >= 0.0), zl_t + base, -1.0)

    rec_part = jnp.dot(rgb1w, mask1, preferred_element_type=f32)
    f_part = jnp.dot(rgb1.astype(bf16), m2, preferred_element_type=f32)

    @pl.when(i == 0)
    def _():
        acc_rec[...] = rec_part
        acc_f[...] = f_part
        acc_cmin[...] = cmax_t
        acc_zl[...] = zl_g

    @pl.when(i > 0)
    def _():
        acc_rec[...] += rec_part
        acc_zl[...] = jnp.maximum(acc_zl[...], zl_g)
        old = acc_cmin[...]
        better = cmax_t < old
        eq = cmax_t == old
        acc_f[...] = jnp.where(better, f_part,
                               acc_f[...] + jnp.where(eq, f_part, 0.0))
        acc_cmin[...] = jnp.minimum(old, cmax_t)

    @pl.when(i == nsteps - 1)
    def _():
        rec, den = acc_rec[0:3, :], acc_rec[3:4, :]
        fs, fc = acc_f[0:3, :], acc_f[3:4, :]
        zl = acc_zl[...]
        nz = den != 0.0
        r1 = jnp.where(nz, rec / jnp.where(nz, den, 1.0), rec)
        empty = jnp.logical_not(nz) & (zl < 0.0)
        out_ref[0:3, :] = jnp.where(empty, fs / fc, r1)
        out_ref[3:4, :] = zl


def _make_zero_fix(M, N, BW, L, NC):
    mesh = plsc.VectorSubcoreMesh(core_axis_name="c", subcore_axis_name="s")

    @functools.partial(
        pl.kernel, mesh=mesh,
        out_type=jax.ShapeDtypeStruct((3 * M,), jnp.float32),
        compiler_params=pltpu.CompilerParams(needs_layout_passes=False),
        scratch_types=[
            pltpu.VMEM((3 * N,), jnp.float32),
            pltpu.VMEM((4, BW), jnp.float32),
            pltpu.VMEM((3 * BW,), jnp.float32),
        ],
    )
    def zero_fix(pre_hbm, rgb_hbm, out_hbm, rgb_v, pre_v, o_v):
        wid = jax.lax.axis_index("s") * NC + jax.lax.axis_index("c")
        base = wid * BW
        pltpu.sync_copy(rgb_hbm, rgb_v)
        pltpu.sync_copy(pre_hbm.at[:, pl.ds(base, BW)], pre_v)
        lane = jax.lax.iota(jnp.int32, L)
        for j in range(BW // L):
            sl = pl.ds(j * L, L)
            zl = pre_v[3, sl]
            hit = zl >= 0.0
            idx3 = jnp.maximum(zl, 0.0).astype(jnp.int32) * 3
            lidx3 = (lane + (j * L)) * 3
            for c in range(3):
                g = plsc.load_gather(rgb_v, [idx3 + c])
                val = jnp.where(hit, g, pre_v[c, sl])
                plsc.store_scatter(o_v, [lidx3 + c], val)
        pltpu.sync_copy(o_v, out_hbm.at[pl.ds(3 * base, 3 * BW)])

    return zero_fix


def kernel(pred_xyz, tgt_xyz, tgt_rgb, search_range):
    del search_range
    M = pred_xyz.shape[0]
    N = tgt_xyz.shape[0]

    pred_t = pred_xyz.T
    rgb1 = jnp.concatenate(
        [tgt_rgb.T, jnp.ones((1, N), jnp.float32)], axis=0)

    pre = pl.pallas_call(
        _recolor_kernel,
        grid=(N // _TN,),
        in_specs=[
            pl.BlockSpec((_TN, 3), lambda i: (i, 0)),
            pl.BlockSpec((4, _TN), lambda i: (0, i)),
            pl.BlockSpec((3, M), lambda i: (0, 0)),
        ],
        out_specs=pl.BlockSpec((4, M), lambda i: (0, 0)),
        out_shape=jax.ShapeDtypeStruct((4, M), jnp.float32),
        scratch_shapes=[
            pltpu.VMEM((4, M), jnp.float32),
            pltpu.VMEM((4, M), jnp.float32),
            pltpu.VMEM((1, M), jnp.float32),
            pltpu.VMEM((1, M), jnp.float32),
        ],
        compiler_params=pltpu.CompilerParams(
            dimension_semantics=("arbitrary",),
        ),
    )(tgt_xyz, rgb1, pred_t)

    info = plsc.get_sparse_core_info()
    nw = info.num_cores * info.num_subcores
    zero_fix = _make_zero_fix(M, N, M // nw, info.num_lanes, info.num_cores)
    return zero_fix(pre, tgt_rgb.reshape(3 * N)).reshape(M, 3)


# speedup vs baseline: 1.0850x; 1.0722x over previous
import functools

import jax
import jax.numpy as jnp
from jax.experimental import pallas as pl
from jax.experimental.pallas import tpu as pltpu
from jax.experimental.pallas import tpu_sc as plsc

_TN = 512


def _recolor_kernel(tgt_ref, rgb1_ref, pred_ref, out_ref,
                    acc_rec, acc_f, acc_cmin, acc_zl):
    i = pl.program_id(0)
    nsteps = pl.num_programs(0)

    t = tgt_ref[...]
    p = pred_ref[...]
    rgb1 = rgb1_ref[...]

    f32 = jnp.float32
    bf16 = jnp.bfloat16
    t_sq = jnp.sum(t * t, axis=1, keepdims=True)
    p_sq = jnp.sum(p * p, axis=0, keepdims=True)
    xp2 = jnp.dot(t.astype(bf16), (p + p).astype(bf16),
                  preferred_element_type=f32)
    d = t_sq + p_sq - xp2

    rowmin = jnp.min(d, axis=1, keepdims=True)
    w = jnp.where(rowmin > 0.0, jax.lax.rsqrt(rowmin), 0.0)
    mask1 = jnp.where(d == rowmin, 1.0, 0.0).astype(bf16)
    rgb1w = (rgb1 * jnp.reshape(w, (1, _TN))).astype(bf16)

    cmax_t = jnp.maximum(jnp.min(d, axis=0, keepdims=True), 0.0)
    m2b = d <= cmax_t
    m2 = jnp.where(m2b, 1.0, 0.0).astype(bf16)

    ridx = jax.lax.broadcasted_iota(jnp.int32, (_TN, 1), 0).astype(f32)
    sel = jnp.where(m2b, ridx, -1.0)
    zl_t = jnp.max(sel, axis=0, keepdims=True)
    base = (i * _TN).astype(f32)
    zl_g = jnp.where((cmax_t <= 0.0) & (zl_t >= 0.0), zl_t + base, -1.0)

    rec_part = jnp.dot(rgb1w, mask1, preferred_element_type=f32)
    f_part = jnp.dot(rgb1.astype(bf16), m2, preferred_element_type=f32)

    @pl.when(i == 0)
    def _():
        acc_rec[...] = rec_part
        acc_f[...] = f_part
        acc_cmin[...] = cmax_t
        acc_zl[...] = zl_g

    @pl.when(i > 0)
    def _():
        acc_rec[...] += rec_part
        acc_zl[...] = jnp.maximum(acc_zl[...], zl_g)
        old = acc_cmin[...]
        better = cmax_t < old
        eq = cmax_t == old
        acc_f[...] = jnp.where(better, f_part,
                               acc_f[...] + jnp.where(eq, f_part, 0.0))
        acc_cmin[...] = jnp.minimum(old, cmax_t)

    @pl.when(i == nsteps - 1)
    def _():
        rec, den = acc_rec[0:3, :], acc_rec[3:4, :]
        fs, fc = acc_f[0:3, :], acc_f[3:4, :]
        zl = acc_zl[...]
        nz = den != 0.0
        r1 = jnp.where(nz, rec / jnp.where(nz, den, 1.0), rec)
        empty = jnp.logical_not(nz) & (zl < 0.0)
        out_ref[0:3, :] = jnp.where(empty, fs / fc, r1)
        out_ref[3:4, :] = zl


def _make_zero_fix(M, N, BW, L, NC):
    mesh = plsc.VectorSubcoreMesh(core_axis_name="c", subcore_axis_name="s")

    @functools.partial(
        pl.kernel, mesh=mesh,
        out_type=jax.ShapeDtypeStruct((3 * M,), jnp.float32),
        compiler_params=pltpu.CompilerParams(needs_layout_passes=False),
        scratch_types=[
            pltpu.VMEM((N,), jnp.float32),
            pltpu.VMEM((N,), jnp.float32),
            pltpu.VMEM((N,), jnp.float32),
            pltpu.VMEM((BW,), jnp.float32),
            pltpu.VMEM((BW,), jnp.float32),
            pltpu.VMEM((BW,), jnp.float32),
            pltpu.VMEM((BW,), jnp.float32),
        ],
    )
    def zero_fix(pre_hbm, rgb_hbm, out_hbm, r_v, g_v, b_v, zl_v, o0, o1, o2):
        wid = jax.lax.axis_index("s") * NC + jax.lax.axis_index("c")
        base = wid * BW
        pltpu.sync_copy(rgb_hbm.at[pl.ds(0, N)], r_v)
        pltpu.sync_copy(rgb_hbm.at[pl.ds(N, N)], g_v)
        pltpu.sync_copy(rgb_hbm.at[pl.ds(2 * N, N)], b_v)
        pltpu.sync_copy(pre_hbm.at[pl.ds(3 * M + base, BW)], zl_v)
        pltpu.sync_copy(pre_hbm.at[pl.ds(base, BW)], o0)
        pltpu.sync_copy(pre_hbm.at[pl.ds(M + base, BW)], o1)
        pltpu.sync_copy(pre_hbm.at[pl.ds(2 * M + base, BW)], o2)
        for j in range(BW // L):
            sl = pl.ds(j * L, L)
            zl = zl_v[sl]
            hit = zl >= 0.0
            idx = jnp.maximum(zl, 0.0).astype(jnp.int32)
            o0[sl] = jnp.where(hit, plsc.load_gather(r_v, [idx]), o0[sl])
            o1[sl] = jnp.where(hit, plsc.load_gather(g_v, [idx]), o1[sl])
            o2[sl] = jnp.where(hit, plsc.load_gather(b_v, [idx]), o2[sl])
        pltpu.sync_copy(o0, out_hbm.at[pl.ds(base, BW)])
        pltpu.sync_copy(o1, out_hbm.at[pl.ds(M + base, BW)])
        pltpu.sync_copy(o2, out_hbm.at[pl.ds(2 * M + base, BW)])

    return zero_fix


def kernel(pred_xyz, tgt_xyz, tgt_rgb, search_range):
    del search_range
    M = pred_xyz.shape[0]
    N = tgt_xyz.shape[0]
    pred_t = pred_xyz.T
    rgb1 = jnp.concatenate(
        [tgt_rgb.T, jnp.ones((1, N), jnp.float32)], axis=0)

    pre = pl.pallas_call(
        _recolor_kernel,
        grid=(N // _TN,),
        in_specs=[
            pl.BlockSpec((_TN, 3), lambda i: (i, 0)),
            pl.BlockSpec((4, _TN), lambda i: (0, i)),
            pl.BlockSpec((3, M), lambda i: (0, 0)),
        ],
        out_specs=pl.BlockSpec((4, M), lambda i: (0, 0)),
        out_shape=jax.ShapeDtypeStruct((4, M), jnp.float32),
        scratch_shapes=[
            pltpu.VMEM((4, M), jnp.float32),
            pltpu.VMEM((4, M), jnp.float32),
            pltpu.VMEM((1, M), jnp.float32),
            pltpu.VMEM((1, M), jnp.float32),
        ],
        compiler_params=pltpu.CompilerParams(
            dimension_semantics=("arbitrary",),
        ),
    )(tgt_xyz, rgb1, pred_t)

    info = plsc.get_sparse_core_info()
    nw = info.num_cores * info.num_subcores
    zero_fix = _make_zero_fix(M, N, M // nw, info.num_lanes, info.num_cores)
    out_flat = zero_fix(pre.reshape(4 * M), rgb1.reshape(4 * N))
    return out_flat.reshape(3, M).T


# speedup vs baseline: 1.1218x; 1.0339x over previous
import functools

import jax
import jax.numpy as jnp
from jax.experimental import pallas as pl
from jax.experimental.pallas import tpu as pltpu
from jax.experimental.pallas import tpu_sc as plsc

_TN = 1024


def _recolor_kernel(tgt_ref, rgb1_ref, pred_ref, out_ref,
                    acc_rec, acc_f, acc_cmin, acc_zl):
    i = pl.program_id(0)
    nsteps = pl.num_programs(0)

    t = tgt_ref[...]
    p = pred_ref[...]
    rgb1 = rgb1_ref[...]

    f32 = jnp.float32
    bf16 = jnp.bfloat16
    t_sq = jnp.sum(t * t, axis=1, keepdims=True)
    p_sq = jnp.sum(p * p, axis=0, keepdims=True)
    xp2 = jnp.dot(t.astype(bf16), (p + p).astype(bf16),
                  preferred_element_type=f32)
    d = t_sq + p_sq - xp2

    rowmin = jnp.min(d, axis=1, keepdims=True)
    w = jnp.where(rowmin > 0.0, jax.lax.rsqrt(rowmin), 0.0)
    mask1 = jnp.where(d == rowmin, 1.0, 0.0).astype(bf16)
    rgb1w = (rgb1 * jnp.reshape(w, (1, _TN))).astype(bf16)

    cmax_t = jnp.maximum(jnp.min(d, axis=0, keepdims=True), 0.0)
    m2b = d <= cmax_t
    m2 = jnp.where(m2b, 1.0, 0.0).astype(bf16)

    ridx = jax.lax.broadcasted_iota(jnp.int32, (_TN, 1), 0).astype(f32)
    sel = jnp.where(m2b, ridx, -1.0)
    zl_t = jnp.max(sel, axis=0, keepdims=True)
    base = (i * _TN).astype(f32)
    zl_g = jnp.where((cmax_t <= 0.0) & (zl_t >= 0.0), zl_t + base, -1.0)

    rec_part = jnp.dot(rgb1w, mask1, preferred_element_type=f32)
    f_part = jnp.dot(rgb1.astype(bf16), m2, preferred_element_type=f32)

    @pl.when(i == 0)
    def _():
        acc_rec[...] = rec_part
        acc_f[...] = f_part
        acc_cmin[...] = cmax_t
        acc_zl[...] = zl_g

    @pl.when(i > 0)
    def _():
        acc_rec[...] += rec_part
        acc_zl[...] = jnp.maximum(acc_zl[...], zl_g)
        old = acc_cmin[...]
        better = cmax_t < old
        eq = cmax_t == old
        acc_f[...] = jnp.where(better, f_part,
                               acc_f[...] + jnp.where(eq, f_part, 0.0))
        acc_cmin[...] = jnp.minimum(old, cmax_t)

    @pl.when(i == nsteps - 1)
    def _():
        rec, den = acc_rec[0:3, :], acc_rec[3:4, :]
        fs, fc = acc_f[0:3, :], acc_f[3:4, :]
        zl = acc_zl[...]
        nz = den != 0.0
        r1 = jnp.where(nz, rec / jnp.where(nz, den, 1.0), rec)
        empty = jnp.logical_not(nz) & (zl < 0.0)
        out_ref[0:3, :] = jnp.where(empty, fs / fc, r1)
        out_ref[3:4, :] = zl


def _make_zero_fix(M, N, BW, L, NC):
    mesh = plsc.VectorSubcoreMesh(core_axis_name="c", subcore_axis_name="s")

    @functools.partial(
        pl.kernel, mesh=mesh,
        out_type=jax.ShapeDtypeStruct((3 * M,), jnp.float32),
        compiler_params=pltpu.CompilerParams(needs_layout_passes=False),
        scratch_types=[
            pltpu.VMEM((N,), jnp.float32),
            pltpu.VMEM((N,), jnp.float32),
            pltpu.VMEM((N,), jnp.float32),
            pltpu.VMEM((BW,), jnp.float32),
            pltpu.VMEM((BW,), jnp.float32),
            pltpu.VMEM((BW,), jnp.float32),
            pltpu.VMEM((BW,), jnp.float32),
        ],
    )
    def zero_fix(pre_hbm, rgb_hbm, out_hbm, r_v, g_v, b_v, zl_v, o0, o1, o2):
        wid = jax.lax.axis_index("s") * NC + jax.lax.axis_index("c")
        base = wid * BW
        pltpu.sync_copy(rgb_hbm.at[pl.ds(0, N)], r_v)
        pltpu.sync_copy(rgb_hbm.at[pl.ds(N, N)], g_v)
        pltpu.sync_copy(rgb_hbm.at[pl.ds(2 * N, N)], b_v)
        pltpu.sync_copy(pre_hbm.at[pl.ds(3 * M + base, BW)], zl_v)
        pltpu.sync_copy(pre_hbm.at[pl.ds(base, BW)], o0)
        pltpu.sync_copy(pre_hbm.at[pl.ds(M + base, BW)], o1)
        pltpu.sync_copy(pre_hbm.at[pl.ds(2 * M + base, BW)], o2)
        for j in range(BW // L):
            sl = pl.ds(j * L, L)
            zl = zl_v[sl]
            hit = zl >= 0.0
            idx = jnp.maximum(zl, 0.0).astype(jnp.int32)
            o0[sl] = jnp.where(hit, plsc.load_gather(r_v, [idx]), o0[sl])
            o1[sl] = jnp.where(hit, plsc.load_gather(g_v, [idx]), o1[sl])
            o2[sl] = jnp.where(hit, plsc.load_gather(b_v, [idx]), o2[sl])
        pltpu.sync_copy(o0, out_hbm.at[pl.ds(base, BW)])
        pltpu.sync_copy(o1, out_hbm.at[pl.ds(M + base, BW)])
        pltpu.sync_copy(o2, out_hbm.at[pl.ds(2 * M + base, BW)])

    return zero_fix


def kernel(pred_xyz, tgt_xyz, tgt_rgb, search_range):
    del search_range
    M = pred_xyz.shape[0]
    N = tgt_xyz.shape[0]
    pred_t = pred_xyz.T
    rgb1 = jnp.concatenate(
        [tgt_rgb.T, jnp.ones((1, N), jnp.float32)], axis=0)

    pre = pl.pallas_call(
        _recolor_kernel,
        grid=(N // _TN,),
        in_specs=[
            pl.BlockSpec((_TN, 3), lambda i: (i, 0)),
            pl.BlockSpec((4, _TN), lambda i: (0, i)),
            pl.BlockSpec((3, M), lambda i: (0, 0)),
        ],
        out_specs=pl.BlockSpec((4, M), lambda i: (0, 0)),
        out_shape=jax.ShapeDtypeStruct((4, M), jnp.float32),
        scratch_shapes=[
            pltpu.VMEM((4, M), jnp.float32),
            pltpu.VMEM((4, M), jnp.float32),
            pltpu.VMEM((1, M), jnp.float32),
            pltpu.VMEM((1, M), jnp.float32),
        ],
        compiler_params=pltpu.CompilerParams(
            dimension_semantics=("arbitrary",),
        ),
    )(tgt_xyz, rgb1, pred_t)

    info = plsc.get_sparse_core_info()
    nw = info.num_cores * info.num_subcores
    zero_fix = _make_zero_fix(M, N, M // nw, info.num_lanes, info.num_cores)
    out_flat = zero_fix(pre.reshape(4 * M), rgb1.reshape(4 * N))
    return out_flat.reshape(3, M).T
